# Initial kernel scaffold; baseline (speedup 1.0000x reference)
import jax
import jax.numpy as jnp
from jax.experimental import pallas as pl


def kernel(P, d_hw, d_error, circuit_edge_pairs, circuit_edge_weights):
    raise NotImplementedError("write your pallas kernel here")


# speedup vs baseline: 15.7205x; 15.7205x over previous
import jax
import jax.numpy as jnp
from jax.experimental import pallas as pl
from jax.experimental.pallas import tpu as pltpu

B, N, E = 256, 128, 512
BC = 32


def _body(p_ref, dhw_ref, derr_ref, i_ref, j_ref, w_ref, out_ref, a_ref, c_ref):
    step = pl.program_id(0)
    nsteps = pl.num_programs(0)

    @pl.when(step == 0)
    def _init():
        a_hw = (dhw_ref[...] == 1.0).astype(jnp.float32)
        fid = jnp.maximum(1.0 - derr_ref[...], 0.0)
        a_ref[...] = a_hw * fid
        cols = jax.lax.broadcasted_iota(jnp.int32, (E, N), 1)
        i = i_ref[0, :][:, None]
        j = j_ref[0, :][:, None]
        w = w_ref[0, :][:, None]
        ioh_w = jnp.where(i == cols, w, 0.0)
        joh = (j == cols).astype(jnp.float32)
        c_ref[...] = jax.lax.dot_general(
            ioh_w, joh, (((0,), (0,)), ((), ())),
            preferred_element_type=jnp.float32)
        out_ref[...] = jnp.zeros_like(out_ref)

    p = p_ref[...]
    x = jax.lax.dot_general(
        p, a_ref[...], (((2,), (0,)), ((), ())),
        preferred_element_type=jnp.float32)
    s = jax.lax.dot_general(
        x, p, (((2,), (2,)), ((0,), (0,))),
        preferred_element_type=jnp.float32)
    total = jnp.sum(c_ref[...] * jnp.sum(s, axis=0))
    out_ref[...] += jnp.reshape(total, (1, 1))

    @pl.when(step == nsteps - 1)
    def _finish():
        sw = jnp.maximum(jnp.sum(w_ref[0, :]), 1e-8)
        out_ref[...] = -out_ref[...] / (B * sw)


def kernel(P, d_hw, d_error, circuit_edge_pairs, circuit_edge_weights):
    edges = circuit_edge_pairs.astype(jnp.int32)
    i_arr = edges[:, 0].reshape(1, E)
    j_arr = edges[:, 1].reshape(1, E)
    w_arr = circuit_edge_weights.reshape(1, E)

    out = pl.pallas_call(
        _body,
        grid=(B // BC,),
        in_specs=[
            pl.BlockSpec((BC, N, N), lambda b: (b, 0, 0)),
            pl.BlockSpec((N, N), lambda b: (0, 0)),
            pl.BlockSpec((N, N), lambda b: (0, 0)),
            pl.BlockSpec((1, E), lambda b: (0, 0)),
            pl.BlockSpec((1, E), lambda b: (0, 0)),
            pl.BlockSpec((1, E), lambda b: (0, 0)),
        ],
        out_specs=pl.BlockSpec((1, 1), lambda b: (0, 0)),
        out_shape=jax.ShapeDtypeStruct((1, 1), jnp.float32),
        scratch_shapes=[
            pltpu.VMEM((N, N), jnp.float32),
            pltpu.VMEM((N, N), jnp.float32),
        ],
    )(P, d_hw, d_error, i_arr, j_arr, w_arr)
    return out.reshape(())
